# baseline (speedup 1.0000x reference)
import functools

import jax
import jax.numpy as jnp
from jax.experimental import pallas as pl
from jax.experimental.pallas import tpu as pltpu

T = 2048
H = 1024
I = 512
E = 16
SI = 1024
TOPK = 2
SCALE = 2.5
CHUNK = 512

NEG_INF = -1e30


def _routing_we(x, gate_w, bias, e):
    logits = jax.lax.dot_general(
        x, gate_w, (((1,), (1,)), ((), ())),
        preferred_element_type=jnp.float32)
    scores = jax.nn.sigmoid(logits)
    sc = scores + bias
    iota = jax.lax.broadcasted_iota(jnp.int32, (T, E), 1)
    m1 = jnp.max(sc, axis=-1, keepdims=True)
    idx1 = jnp.min(jnp.where(sc == m1, iota, E), axis=-1, keepdims=True)
    sc2 = jnp.where(iota == idx1, NEG_INF, sc)
    m2 = jnp.max(sc2, axis=-1, keepdims=True)
    idx2 = jnp.min(jnp.where(sc2 == m2, iota, E), axis=-1, keepdims=True)
    w1 = jnp.sum(jnp.where(iota == idx1, scores, 0.0), axis=-1, keepdims=True)
    w2 = jnp.sum(jnp.where(iota == idx2, scores, 0.0), axis=-1, keepdims=True)
    denom = w1 + w2 + 1e-20
    w1 = w1 / denom * SCALE
    w2 = w2 / denom * SCALE
    we = jnp.where(idx1 == e, w1, 0.0) + jnp.where(idx2 == e, w2, 0.0)
    return we


def _moe_kernel(x_ref, gate_w_ref, bias_ref, gup_ref, down_ref,
                sgw_ref, suw_ref, sdw_ref, out_ref):
    e = pl.program_id(0)
    x = x_ref[...]
    we = _routing_we(x, gate_w_ref[...], bias_ref[...], e)

    gup = gup_ref[0]
    dwn = down_ref[0]
    xb = x.astype(jnp.bfloat16)

    @pl.when(e == 0)
    def _init():
        for c in range(T // CHUNK):
            xc = xb[c * CHUNK:(c + 1) * CHUNK]
            sg = jax.lax.dot_general(xc, sgw_ref[...], (((1,), (1,)), ((), ())),
                                     preferred_element_type=jnp.float32)
            su = jax.lax.dot_general(xc, suw_ref[...], (((1,), (1,)), ((), ())),
                                     preferred_element_type=jnp.float32)
            h = (jax.nn.silu(sg) * su).astype(jnp.bfloat16)
            y = jax.lax.dot_general(h, sdw_ref[...], (((1,), (1,)), ((), ())),
                                    preferred_element_type=jnp.float32)
            out_ref[c * CHUNK:(c + 1) * CHUNK, :] = y

    for c in range(T // CHUNK):
        xc = xb[c * CHUNK:(c + 1) * CHUNK]
        gu = jax.lax.dot_general(xc, gup, (((1,), (1,)), ((), ())),
                                 preferred_element_type=jnp.float32)
        h = (jax.nn.silu(gu[:, :I]) * gu[:, I:]).astype(jnp.bfloat16)
        y = jax.lax.dot_general(h, dwn, (((1,), (1,)), ((), ())),
                                preferred_element_type=jnp.float32)
        out_ref[c * CHUNK:(c + 1) * CHUNK, :] += \
            we[c * CHUNK:(c + 1) * CHUNK] * y


def kernel(hidden_states, gate_w, e_score_correction_bias, gate_up_proj,
           down_proj, shared_gate_w, shared_up_w, shared_down_w):
    x = hidden_states.reshape(T, H)
    bias = e_score_correction_bias.reshape(1, E)
    gate_up_proj = gate_up_proj.astype(jnp.bfloat16)
    down_proj = down_proj.astype(jnp.bfloat16)
    shared_gate_w = shared_gate_w.astype(jnp.bfloat16)
    shared_up_w = shared_up_w.astype(jnp.bfloat16)
    shared_down_w = shared_down_w.astype(jnp.bfloat16)

    out = pl.pallas_call(
        _moe_kernel,
        grid=(E,),
        in_specs=[
            pl.BlockSpec((T, H), lambda e: (0, 0)),
            pl.BlockSpec((E, H), lambda e: (0, 0)),
            pl.BlockSpec((1, E), lambda e: (0, 0)),
            pl.BlockSpec((1, 2 * I, H), lambda e: (e, 0, 0)),
            pl.BlockSpec((1, H, I), lambda e: (e, 0, 0)),
            pl.BlockSpec((SI, H), lambda e: (0, 0)),
            pl.BlockSpec((SI, H), lambda e: (0, 0)),
            pl.BlockSpec((H, SI), lambda e: (0, 0)),
        ],
        out_specs=pl.BlockSpec((T, H), lambda e: (0, 0)),
        out_shape=jax.ShapeDtypeStruct((T, H), jnp.float32),
        compiler_params=pltpu.CompilerParams(
            dimension_semantics=("arbitrary",),
        ),
    )(x, gate_w, bias, gate_up_proj, down_proj,
      shared_gate_w, shared_up_w, shared_down_w)

    return out.reshape(hidden_states.shape)


# speedup vs baseline: 1.1518x; 1.1518x over previous
import functools

import jax
import jax.numpy as jnp
from jax import lax
from jax.experimental import pallas as pl
from jax.experimental.pallas import tpu as pltpu
from jax.experimental.pallas import tpu_sc as plsc

T = 2048
H = 1024
I = 512
E = 16
SI = 1024
SCALE = 2.5
NEG_INF = -1e30

B = 256
NBLK = 32
PAD_LEN = NBLK * B
RCH = 256
SHC = T // NBLK

NC, NS = 2, 16
NW = NC * NS
TPT = T // NW
CT = 32


def _router_kernel(x_ref, gate_w_ref, bias_ref,
                   pos1_ref, pos2_ref, w1_ref, w2_ref, be_ref, valid_ref):
    x = x_ref[...]
    logits = lax.dot_general(x, gate_w_ref[...], (((1,), (1,)), ((), ())),
                             preferred_element_type=jnp.float32)
    scores = jax.nn.sigmoid(logits)
    sc = scores + bias_ref[...]
    iota = lax.broadcasted_iota(jnp.int32, (T, E), 1)
    m1 = jnp.max(sc, axis=-1, keepdims=True)
    idx1 = jnp.min(jnp.where(sc == m1, iota, E), axis=-1, keepdims=True)
    sc2 = jnp.where(iota == idx1, NEG_INF, sc)
    m2 = jnp.max(sc2, axis=-1, keepdims=True)
    idx2 = jnp.min(jnp.where(sc2 == m2, iota, E), axis=-1, keepdims=True)
    w1 = jnp.sum(jnp.where(iota == idx1, scores, 0.0), axis=-1, keepdims=True)
    w2 = jnp.sum(jnp.where(iota == idx2, scores, 0.0), axis=-1, keepdims=True)
    denom = w1 + w2 + 1e-20
    w1_ref[...] = w1 / denom * SCALE
    w2_ref[...] = w2 / denom * SCALE

    oh1 = (iota == idx1).astype(jnp.float32)
    oh2 = (iota == idx2).astype(jnp.float32)

    tri = (lax.broadcasted_iota(jnp.int32, (RCH, RCH), 0) >
           lax.broadcasted_iota(jnp.int32, (RCH, RCH), 1)).astype(jnp.float32)
    off = jnp.zeros((1, E), jnp.float32)
    ranks = []
    for oh in (oh1, oh2):
        rs = []
        for c in range(T // RCH):
            ohc = oh[c * RCH:(c + 1) * RCH]
            exc = lax.dot_general(tri, ohc, (((1,), (0,)), ((), ())),
                                  preferred_element_type=jnp.float32) + off
            rs.append(jnp.sum(exc * ohc, axis=-1, keepdims=True))
            off = off + jnp.sum(ohc, axis=0, keepdims=True)
        ranks.append(jnp.concatenate(rs, axis=0))
    counts = off

    pc_pad = jnp.ceil(counts / B) * B
    mstrict = (lax.broadcasted_iota(jnp.int32, (E, E), 0) <
               lax.broadcasted_iota(jnp.int32, (E, E), 1)).astype(jnp.float32)
    pad_off = lax.dot_general(pc_pad, mstrict, (((1,), (0,)), ((), ())),
                              preferred_element_type=jnp.float32)
    total_pad = jnp.sum(pc_pad, axis=-1, keepdims=True)

    sel1 = jnp.sum(oh1 * pad_off, axis=-1, keepdims=True)
    sel2 = jnp.sum(oh2 * pad_off, axis=-1, keepdims=True)
    pos1_ref[...] = (sel1 + ranks[0]).astype(jnp.int32)
    pos2_ref[...] = (sel2 + ranks[1]).astype(jnp.int32)

    bb = lax.broadcasted_iota(jnp.int32, (NBLK, E), 0).astype(jnp.float32) * B
    le = (jnp.broadcast_to(pad_off, (NBLK, E)) <= bb).astype(jnp.float32)
    be_ref[...] = (jnp.sum(le, axis=-1, keepdims=True) - 1.0).astype(jnp.int32)
    bb0 = lax.broadcasted_iota(jnp.int32, (NBLK, 1), 0).astype(jnp.float32) * B
    valid_ref[...] = (bb0 < total_pad).astype(jnp.int32)


def _router(x, gate_w, bias):
    return pl.pallas_call(
        _router_kernel,
        grid=(1,),
        in_specs=[
            pl.BlockSpec((T, H), lambda i: (0, 0)),
            pl.BlockSpec((E, H), lambda i: (0, 0)),
            pl.BlockSpec((1, E), lambda i: (0, 0)),
        ],
        out_specs=[
            pl.BlockSpec((T, 1), lambda i: (0, 0)),
            pl.BlockSpec((T, 1), lambda i: (0, 0)),
            pl.BlockSpec((T, 1), lambda i: (0, 0)),
            pl.BlockSpec((T, 1), lambda i: (0, 0)),
            pl.BlockSpec((NBLK, 1), lambda i: (0, 0)),
            pl.BlockSpec((NBLK, 1), lambda i: (0, 0)),
        ],
        out_shape=[
            jax.ShapeDtypeStruct((T, 1), jnp.int32),
            jax.ShapeDtypeStruct((T, 1), jnp.int32),
            jax.ShapeDtypeStruct((T, 1), jnp.float32),
            jax.ShapeDtypeStruct((T, 1), jnp.float32),
            jax.ShapeDtypeStruct((NBLK, 1), jnp.int32),
            jax.ShapeDtypeStruct((NBLK, 1), jnp.int32),
        ],
    )(x, gate_w, bias)


def _dispatch_body(x_hbm, pos1_hbm, pos2_hbm, w1_hbm, w2_hbm,
                   xs_hbm, ws_hbm,
                   rows_v, idx1_v, idx2_v, wv1, wv2, sem):
    wid = lax.axis_index("s") * NC + lax.axis_index("c")
    base = wid * TPT
    pltpu.sync_copy(x_hbm.at[pl.ds(base, TPT), :], rows_v)
    pltpu.sync_copy(pos1_hbm.at[pl.ds(base, TPT)], idx1_v)
    pltpu.sync_copy(pos2_hbm.at[pl.ds(base, TPT)], idx2_v)
    pltpu.sync_copy(w1_hbm.at[pl.ds(base, TPT)], wv1)
    pltpu.sync_copy(w2_hbm.at[pl.ds(base, TPT)], wv2)
    c1 = pltpu.async_copy(rows_v, xs_hbm.at[idx1_v], sem)
    c2 = pltpu.async_copy(rows_v, xs_hbm.at[idx2_v], sem)
    c1.wait()
    c2.wait()
    c3 = pltpu.async_copy(wv1, ws_hbm.at[idx1_v], sem)
    c4 = pltpu.async_copy(wv2, ws_hbm.at[idx2_v], sem)
    c3.wait()
    c4.wait()


def _dispatch(x, pos1, pos2, w1, w2):
    mesh = plsc.VectorSubcoreMesh(core_axis_name="c", subcore_axis_name="s",
                                  num_cores=NC, num_subcores=NS)
    fn = pl.kernel(
        _dispatch_body,
        out_type=[
            jax.ShapeDtypeStruct((PAD_LEN, H), jnp.float32),
            jax.ShapeDtypeStruct((PAD_LEN,), jnp.float32),
        ],
        mesh=mesh,
        scratch_types=[
            pltpu.VMEM((TPT, H), jnp.float32),
            pltpu.VMEM((TPT,), jnp.int32),
            pltpu.VMEM((TPT,), jnp.int32),
            pltpu.VMEM((TPT,), jnp.float32),
            pltpu.VMEM((TPT,), jnp.float32),
            pltpu.SemaphoreType.DMA,
        ],
    )
    return fn(x, pos1, pos2, w1, w2)


def _grouped_kernel(be_ref, valid_ref, xs_ref, gup_ref, down_ref, ws_ref,
                    x_ref, sgw_ref, suw_ref, sdw_ref, yw_ref, so_ref):
    b = pl.program_id(0)
    xc = x_ref[...]
    sg = lax.dot_general(xc, sgw_ref[...], (((1,), (1,)), ((), ())),
                         preferred_element_type=jnp.float32)
    su = lax.dot_general(xc, suw_ref[...], (((1,), (1,)), ((), ())),
                         preferred_element_type=jnp.float32)
    hsh = jax.nn.silu(sg) * su
    so_ref[...] = lax.dot_general(hsh, sdw_ref[...], (((1,), (1,)), ((), ())),
                                  preferred_element_type=jnp.float32)

    @pl.when(valid_ref[b] > 0)
    def _():
        xb = xs_ref[...]
        gu = lax.dot_general(xb, gup_ref[0], (((1,), (1,)), ((), ())),
                             preferred_element_type=jnp.float32)
        h = jax.nn.silu(gu[:, :I]) * gu[:, I:]
        y = lax.dot_general(h, down_ref[0], (((1,), (1,)), ((), ())),
                            preferred_element_type=jnp.float32)
        w = ws_ref[0, 0, :].reshape(B, 1)
        yw_ref[...] = w * y


def _grouped(be, valid, xs, gup, down, ws3, x, sgw, suw, sdw):
    grid_spec = pltpu.PrefetchScalarGridSpec(
        num_scalar_prefetch=2,
        grid=(NBLK,),
        in_specs=[
            pl.BlockSpec((B, H), lambda b, be_r, v_r: (b, 0)),
            pl.BlockSpec((1, 2 * I, H), lambda b, be_r, v_r: (be_r[b], 0, 0)),
            pl.BlockSpec((1, H, I), lambda b, be_r, v_r: (be_r[b], 0, 0)),
            pl.BlockSpec((1, 1, B), lambda b, be_r, v_r: (b, 0, 0)),
            pl.BlockSpec((SHC, H), lambda b, be_r, v_r: (b, 0)),
            pl.BlockSpec((SI, H), lambda b, be_r, v_r: (0, 0)),
            pl.BlockSpec((SI, H), lambda b, be_r, v_r: (0, 0)),
            pl.BlockSpec((H, SI), lambda b, be_r, v_r: (0, 0)),
        ],
        out_specs=[
            pl.BlockSpec((B, H), lambda b, be_r, v_r: (b, 0)),
            pl.BlockSpec((SHC, H), lambda b, be_r, v_r: (b, 0)),
        ],
    )
    return pl.pallas_call(
        _grouped_kernel,
        grid_spec=grid_spec,
        out_shape=[
            jax.ShapeDtypeStruct((PAD_LEN, H), jnp.float32),
            jax.ShapeDtypeStruct((T, H), jnp.float32),
        ],
        compiler_params=pltpu.CompilerParams(
            dimension_semantics=("arbitrary",),
        ),
    )(be, valid, xs, gup, down, ws3, x, sgw, suw, sdw)


def _combine_body(yw_hbm, so_hbm, pos1_hbm, pos2_hbm, out_hbm,
                  y1_v, y2_v, sh_v, i1_v, i2_v, sem):
    wid = lax.axis_index("s") * NC + lax.axis_index("c")
    for ch in range(TPT // CT):
        tbase = wid * TPT + ch * CT
        pltpu.sync_copy(pos1_hbm.at[pl.ds(tbase, CT)], i1_v)
        pltpu.sync_copy(pos2_hbm.at[pl.ds(tbase, CT)], i2_v)
        g1 = pltpu.async_copy(yw_hbm.at[i1_v], y1_v, sem)
        g2 = pltpu.async_copy(yw_hbm.at[i2_v], y2_v, sem)
        pltpu.sync_copy(so_hbm.at[pl.ds(tbase, CT), :], sh_v)
        g1.wait()
        g2.wait()

        def body(i, _):
            r = lax.shift_right_logical(i, 6)
            col = pl.multiple_of(lax.shift_left(jnp.bitwise_and(i, 63), 4), 16)
            sl = pl.ds(col, 16)
            y1_v[r, sl] = y1_v[r, sl] + y2_v[r, sl] + sh_v[r, sl]
            return 0

        lax.fori_loop(0, CT * (H // 16), body, 0)
        pltpu.sync_copy(y1_v, out_hbm.at[pl.ds(tbase, CT), :])


def _combine(yw, so, pos1, pos2):
    mesh = plsc.VectorSubcoreMesh(core_axis_name="c", subcore_axis_name="s",
                                  num_cores=NC, num_subcores=NS)
    fn = pl.kernel(
        _combine_body,
        out_type=jax.ShapeDtypeStruct((T, H), jnp.float32),
        mesh=mesh,
        scratch_types=[
            pltpu.VMEM((CT, H), jnp.float32),
            pltpu.VMEM((CT, H), jnp.float32),
            pltpu.VMEM((CT, H), jnp.float32),
            pltpu.VMEM((CT,), jnp.int32),
            pltpu.VMEM((CT,), jnp.int32),
            pltpu.SemaphoreType.DMA,
        ],
    )
    return fn(yw, so, pos1, pos2)


def kernel(hidden_states, gate_w, e_score_correction_bias, gate_up_proj,
           down_proj, shared_gate_w, shared_up_w, shared_down_w):
    x = hidden_states.reshape(T, H)
    bias = e_score_correction_bias.reshape(1, E)

    pos1, pos2, w1, w2, be, valid = _router(x, gate_w, bias)
    pos1f = pos1.reshape(T)
    pos2f = pos2.reshape(T)

    xs, ws = _dispatch(x, pos1f, pos2f, w1.reshape(T), w2.reshape(T))

    yw, so = _grouped(be.reshape(NBLK), valid.reshape(NBLK), xs,
                      gate_up_proj, down_proj, ws.reshape(NBLK, 1, B),
                      x, shared_gate_w, shared_up_w, shared_down_w)

    out = _combine(yw, so, pos1f, pos2f)
    return out.reshape(hidden_states.shape)


# speedup vs baseline: 1.2215x; 1.0605x over previous
import functools

import jax
import jax.numpy as jnp
from jax import lax
from jax.experimental import pallas as pl
from jax.experimental.pallas import tpu as pltpu
from jax.experimental.pallas import tpu_sc as plsc

T = 2048
H = 1024
I = 512
E = 16
SI = 1024
SCALE = 2.5
NEG_INF = -1e30

B = 256
NBLK = 32
PAD_LEN = NBLK * B
RCH = 256
SHC = T // NBLK

NC, NS = 2, 16
NW = NC * NS
TPT = T // NW
CT = 32


def _router_kernel(x_ref, gate_w_ref, bias_ref,
                   pos1_ref, pos2_ref, w1_ref, w2_ref, be_ref, valid_ref):
    x = x_ref[...]
    logits = lax.dot_general(x, gate_w_ref[...], (((1,), (1,)), ((), ())),
                             preferred_element_type=jnp.float32)
    scores = jax.nn.sigmoid(logits)
    sc = scores + bias_ref[...]
    iota = lax.broadcasted_iota(jnp.int32, (T, E), 1)
    m1 = jnp.max(sc, axis=-1, keepdims=True)
    idx1 = jnp.min(jnp.where(sc == m1, iota, E), axis=-1, keepdims=True)
    sc2 = jnp.where(iota == idx1, NEG_INF, sc)
    m2 = jnp.max(sc2, axis=-1, keepdims=True)
    idx2 = jnp.min(jnp.where(sc2 == m2, iota, E), axis=-1, keepdims=True)
    w1 = jnp.sum(jnp.where(iota == idx1, scores, 0.0), axis=-1, keepdims=True)
    w2 = jnp.sum(jnp.where(iota == idx2, scores, 0.0), axis=-1, keepdims=True)
    denom = w1 + w2 + 1e-20
    w1_ref[...] = w1 / denom * SCALE
    w2_ref[...] = w2 / denom * SCALE

    oh1 = (iota == idx1).astype(jnp.float32)
    oh2 = (iota == idx2).astype(jnp.float32)

    tri = (lax.broadcasted_iota(jnp.int32, (RCH, RCH), 0) >
           lax.broadcasted_iota(jnp.int32, (RCH, RCH), 1)).astype(jnp.float32)
    off = jnp.zeros((1, E), jnp.float32)
    ranks = []
    for oh in (oh1, oh2):
        rs = []
        for c in range(T // RCH):
            ohc = oh[c * RCH:(c + 1) * RCH]
            exc = lax.dot_general(tri, ohc, (((1,), (0,)), ((), ())),
                                  preferred_element_type=jnp.float32) + off
            rs.append(jnp.sum(exc * ohc, axis=-1, keepdims=True))
            off = off + jnp.sum(ohc, axis=0, keepdims=True)
        ranks.append(jnp.concatenate(rs, axis=0))
    counts = off

    pc_pad = jnp.ceil(counts / B) * B
    mstrict = (lax.broadcasted_iota(jnp.int32, (E, E), 0) <
               lax.broadcasted_iota(jnp.int32, (E, E), 1)).astype(jnp.float32)
    pad_off = lax.dot_general(pc_pad, mstrict, (((1,), (0,)), ((), ())),
                              preferred_element_type=jnp.float32)
    total_pad = jnp.sum(pc_pad, axis=-1, keepdims=True)

    sel1 = jnp.sum(oh1 * pad_off, axis=-1, keepdims=True)
    sel2 = jnp.sum(oh2 * pad_off, axis=-1, keepdims=True)
    pos1_ref[...] = (sel1 + ranks[0]).astype(jnp.int32)
    pos2_ref[...] = (sel2 + ranks[1]).astype(jnp.int32)

    bb = lax.broadcasted_iota(jnp.int32, (NBLK, E), 0).astype(jnp.float32) * B
    le = (jnp.broadcast_to(pad_off, (NBLK, E)) <= bb).astype(jnp.float32)
    be_ref[...] = (jnp.sum(le, axis=-1, keepdims=True) - 1.0).astype(jnp.int32)
    bb0 = lax.broadcasted_iota(jnp.int32, (NBLK, 1), 0).astype(jnp.float32) * B
    valid_ref[...] = (bb0 < total_pad).astype(jnp.int32)


def _router(x, gate_w, bias):
    return pl.pallas_call(
        _router_kernel,
        grid=(1,),
        in_specs=[
            pl.BlockSpec((T, H), lambda i: (0, 0)),
            pl.BlockSpec((E, H), lambda i: (0, 0)),
            pl.BlockSpec((1, E), lambda i: (0, 0)),
        ],
        out_specs=[
            pl.BlockSpec((T, 1), lambda i: (0, 0)),
            pl.BlockSpec((T, 1), lambda i: (0, 0)),
            pl.BlockSpec((T, 1), lambda i: (0, 0)),
            pl.BlockSpec((T, 1), lambda i: (0, 0)),
            pl.BlockSpec((NBLK, 1), lambda i: (0, 0)),
            pl.BlockSpec((NBLK, 1), lambda i: (0, 0)),
        ],
        out_shape=[
            jax.ShapeDtypeStruct((T, 1), jnp.int32),
            jax.ShapeDtypeStruct((T, 1), jnp.int32),
            jax.ShapeDtypeStruct((T, 1), jnp.float32),
            jax.ShapeDtypeStruct((T, 1), jnp.float32),
            jax.ShapeDtypeStruct((NBLK, 1), jnp.int32),
            jax.ShapeDtypeStruct((NBLK, 1), jnp.int32),
        ],
    )(x, gate_w, bias)


def _dispatch_body(x_hbm, pos1_hbm, pos2_hbm, w1_hbm, w2_hbm,
                   xs_hbm, ws_hbm,
                   rows_v, idx1_v, idx2_v, wv1, wv2, sem):
    wid = lax.axis_index("s") * NC + lax.axis_index("c")
    base = wid * TPT
    pltpu.sync_copy(pos1_hbm.at[pl.ds(base, TPT)], idx1_v)
    pltpu.sync_copy(pos2_hbm.at[pl.ds(base, TPT)], idx2_v)
    pltpu.sync_copy(w1_hbm.at[pl.ds(base, TPT)], wv1)
    pltpu.sync_copy(w2_hbm.at[pl.ds(base, TPT)], wv2)
    c3 = pltpu.async_copy(wv1, ws_hbm.at[idx1_v], sem)
    c4 = pltpu.async_copy(wv2, ws_hbm.at[idx2_v], sem)
    pltpu.sync_copy(x_hbm.at[pl.ds(base, TPT), :], rows_v)
    c1 = pltpu.async_copy(rows_v, xs_hbm.at[idx1_v], sem)
    c2 = pltpu.async_copy(rows_v, xs_hbm.at[idx2_v], sem)
    c3.wait()
    c4.wait()
    c1.wait()
    c2.wait()


def _dispatch(x, pos1, pos2, w1, w2):
    mesh = plsc.VectorSubcoreMesh(core_axis_name="c", subcore_axis_name="s",
                                  num_cores=NC, num_subcores=NS)
    fn = pl.kernel(
        _dispatch_body,
        out_type=[
            jax.ShapeDtypeStruct((PAD_LEN, H), jnp.float32),
            jax.ShapeDtypeStruct((PAD_LEN,), jnp.float32),
        ],
        mesh=mesh,
        scratch_types=[
            pltpu.VMEM((TPT, H), jnp.float32),
            pltpu.VMEM((TPT,), jnp.int32),
            pltpu.VMEM((TPT,), jnp.int32),
            pltpu.VMEM((TPT,), jnp.float32),
            pltpu.VMEM((TPT,), jnp.float32),
            pltpu.SemaphoreType.DMA,
        ],
    )
    return fn(x, pos1, pos2, w1, w2)


def _grouped_kernel(be_ref, valid_ref, xs_ref, gup_ref, down_ref, ws_ref,
                    x_ref, sgw_ref, suw_ref, sdw_ref, yw_ref, so_ref):
    b = pl.program_id(0)
    xc = x_ref[...]
    sg = lax.dot_general(xc, sgw_ref[...], (((1,), (1,)), ((), ())),
                         preferred_element_type=jnp.float32)
    su = lax.dot_general(xc, suw_ref[...], (((1,), (1,)), ((), ())),
                         preferred_element_type=jnp.float32)
    hsh = jax.nn.silu(sg) * su
    so_ref[...] = lax.dot_general(hsh, sdw_ref[...], (((1,), (1,)), ((), ())),
                                  preferred_element_type=jnp.float32)

    @pl.when(valid_ref[b] > 0)
    def _():
        xb = xs_ref[...].astype(jnp.bfloat16)
        gup = gup_ref[0].astype(jnp.bfloat16)
        gu = lax.dot_general(xb, gup, (((1,), (1,)), ((), ())),
                             preferred_element_type=jnp.float32)
        h = (jax.nn.silu(gu[:, :I]) * gu[:, I:]).astype(jnp.bfloat16)
        dwn = down_ref[0].astype(jnp.bfloat16)
        y = lax.dot_general(h, dwn, (((1,), (1,)), ((), ())),
                            preferred_element_type=jnp.float32)
        w = ws_ref[0, 0, :].reshape(B, 1)
        yw_ref[...] = w * y


def _grouped(be, valid, xs, gup, down, ws3, x, sgw, suw, sdw):
    grid_spec = pltpu.PrefetchScalarGridSpec(
        num_scalar_prefetch=2,
        grid=(NBLK,),
        in_specs=[
            pl.BlockSpec((B, H), lambda b, be_r, v_r: (b, 0)),
            pl.BlockSpec((1, 2 * I, H), lambda b, be_r, v_r: (be_r[b], 0, 0)),
            pl.BlockSpec((1, H, I), lambda b, be_r, v_r: (be_r[b], 0, 0)),
            pl.BlockSpec((1, 1, B), lambda b, be_r, v_r: (b, 0, 0)),
            pl.BlockSpec((SHC, H), lambda b, be_r, v_r: (b, 0)),
            pl.BlockSpec((SI, H), lambda b, be_r, v_r: (0, 0)),
            pl.BlockSpec((SI, H), lambda b, be_r, v_r: (0, 0)),
            pl.BlockSpec((H, SI), lambda b, be_r, v_r: (0, 0)),
        ],
        out_specs=[
            pl.BlockSpec((B, H), lambda b, be_r, v_r: (b, 0)),
            pl.BlockSpec((SHC, H), lambda b, be_r, v_r: (b, 0)),
        ],
    )
    return pl.pallas_call(
        _grouped_kernel,
        grid_spec=grid_spec,
        out_shape=[
            jax.ShapeDtypeStruct((PAD_LEN, H), jnp.float32),
            jax.ShapeDtypeStruct((T, H), jnp.float32),
        ],
        compiler_params=pltpu.CompilerParams(
            dimension_semantics=("arbitrary",),
        ),
    )(be, valid, xs, gup, down, ws3, x, sgw, suw, sdw)


def _combine_body(yw_hbm, so_hbm, pos1_hbm, pos2_hbm, out_hbm,
                  y1_v, y2_v, sh_v, i1_v, i2_v, sem):
    wid = lax.axis_index("s") * NC + lax.axis_index("c")
    for ch in range(TPT // CT):
        tbase = wid * TPT + ch * CT
        pltpu.sync_copy(pos1_hbm.at[pl.ds(tbase, CT)], i1_v)
        pltpu.sync_copy(pos2_hbm.at[pl.ds(tbase, CT)], i2_v)
        g1 = pltpu.async_copy(yw_hbm.at[i1_v], y1_v, sem)
        g2 = pltpu.async_copy(yw_hbm.at[i2_v], y2_v, sem)
        pltpu.sync_copy(so_hbm.at[pl.ds(tbase, CT), :], sh_v)
        g1.wait()
        g2.wait()

        def body(r, _):
            for c in range(H // 16):
                sl = pl.ds(c * 16, 16)
                y1_v[r, sl] = y1_v[r, sl] + y2_v[r, sl] + sh_v[r, sl]
            return 0

        lax.fori_loop(0, CT, body, 0)
        pltpu.sync_copy(y1_v, out_hbm.at[pl.ds(tbase, CT), :])


def _combine(yw, so, pos1, pos2):
    mesh = plsc.VectorSubcoreMesh(core_axis_name="c", subcore_axis_name="s",
                                  num_cores=NC, num_subcores=NS)
    fn = pl.kernel(
        _combine_body,
        out_type=jax.ShapeDtypeStruct((T, H), jnp.float32),
        mesh=mesh,
        scratch_types=[
            pltpu.VMEM((CT, H), jnp.float32),
            pltpu.VMEM((CT, H), jnp.float32),
            pltpu.VMEM((CT, H), jnp.float32),
            pltpu.VMEM((CT,), jnp.int32),
            pltpu.VMEM((CT,), jnp.int32),
            pltpu.SemaphoreType.DMA,
        ],
    )
    return fn(yw, so, pos1, pos2)


def kernel(hidden_states, gate_w, e_score_correction_bias, gate_up_proj,
           down_proj, shared_gate_w, shared_up_w, shared_down_w):
    x = hidden_states.reshape(T, H)
    bias = e_score_correction_bias.reshape(1, E)

    pos1, pos2, w1, w2, be, valid = _router(x, gate_w, bias)
    pos1f = pos1.reshape(T)
    pos2f = pos2.reshape(T)

    xs, ws = _dispatch(x, pos1f, pos2f, w1.reshape(T), w2.reshape(T))

    yw, so = _grouped(be.reshape(NBLK), valid.reshape(NBLK), xs,
                      gate_up_proj, down_proj, ws.reshape(NBLK, 1, B),
                      x, shared_gate_w, shared_up_w, shared_down_w)

    out = _combine(yw, so, pos1f, pos2f)
    return out.reshape(hidden_states.shape)


# speedup vs baseline: 1.4517x; 1.1884x over previous
import functools

import jax
import jax.numpy as jnp
from jax import lax
from jax.experimental import pallas as pl
from jax.experimental.pallas import tpu as pltpu
from jax.experimental.pallas import tpu_sc as plsc

T = 2048
H = 1024
I = 512
E = 16
SI = 1024
SCALE = 2.5
NEG_INF = -1e30

B = 256
NBLK = 32
PAD_LEN = NBLK * B
RCH = 256
SHC = T // NBLK

NC, NS = 2, 16
NW = NC * NS
TPT = T // NW
CT = 32


def _router_kernel(x_ref, gate_w_ref, bias_ref,
                   pos1_ref, pos2_ref, w1_ref, w2_ref, be_ref, valid_ref):
    x = x_ref[...]
    logits = lax.dot_general(x, gate_w_ref[...], (((1,), (1,)), ((), ())),
                             preferred_element_type=jnp.float32)
    scores = jax.nn.sigmoid(logits)
    sc = scores + bias_ref[...]
    iota = lax.broadcasted_iota(jnp.int32, (T, E), 1)
    m1 = jnp.max(sc, axis=-1, keepdims=True)
    idx1 = jnp.min(jnp.where(sc == m1, iota, E), axis=-1, keepdims=True)
    sc2 = jnp.where(iota == idx1, NEG_INF, sc)
    m2 = jnp.max(sc2, axis=-1, keepdims=True)
    idx2 = jnp.min(jnp.where(sc2 == m2, iota, E), axis=-1, keepdims=True)
    w1 = jnp.sum(jnp.where(iota == idx1, scores, 0.0), axis=-1, keepdims=True)
    w2 = jnp.sum(jnp.where(iota == idx2, scores, 0.0), axis=-1, keepdims=True)
    denom = w1 + w2 + 1e-20
    w1_ref[...] = w1 / denom * SCALE
    w2_ref[...] = w2 / denom * SCALE

    oh1 = (iota == idx1).astype(jnp.float32)
    oh2 = (iota == idx2).astype(jnp.float32)

    tri = (lax.broadcasted_iota(jnp.int32, (RCH, RCH), 0) >
           lax.broadcasted_iota(jnp.int32, (RCH, RCH), 1)).astype(jnp.float32)
    off = jnp.zeros((1, E), jnp.float32)
    ranks = []
    for oh in (oh1, oh2):
        rs = []
        for c in range(T // RCH):
            ohc = oh[c * RCH:(c + 1) * RCH]
            exc = lax.dot_general(tri, ohc, (((1,), (0,)), ((), ())),
                                  preferred_element_type=jnp.float32) + off
            rs.append(jnp.sum(exc * ohc, axis=-1, keepdims=True))
            off = off + jnp.sum(ohc, axis=0, keepdims=True)
        ranks.append(jnp.concatenate(rs, axis=0))
    counts = off

    pc_pad = jnp.ceil(counts / B) * B
    mstrict = (lax.broadcasted_iota(jnp.int32, (E, E), 0) <
               lax.broadcasted_iota(jnp.int32, (E, E), 1)).astype(jnp.float32)
    pad_off = lax.dot_general(pc_pad, mstrict, (((1,), (0,)), ((), ())),
                              preferred_element_type=jnp.float32)
    total_pad = jnp.sum(pc_pad, axis=-1, keepdims=True)

    sel1 = jnp.sum(oh1 * pad_off, axis=-1, keepdims=True)
    sel2 = jnp.sum(oh2 * pad_off, axis=-1, keepdims=True)
    pos1_ref[...] = (sel1 + ranks[0]).astype(jnp.int32)
    pos2_ref[...] = (sel2 + ranks[1]).astype(jnp.int32)

    bb = lax.broadcasted_iota(jnp.int32, (NBLK, E), 0).astype(jnp.float32) * B
    le = (jnp.broadcast_to(pad_off, (NBLK, E)) <= bb).astype(jnp.float32)
    be_ref[...] = (jnp.sum(le, axis=-1, keepdims=True) - 1.0).astype(jnp.int32)
    bb0 = lax.broadcasted_iota(jnp.int32, (NBLK, 1), 0).astype(jnp.float32) * B
    valid_ref[...] = (bb0 < total_pad).astype(jnp.int32)


def _router(x, gate_w, bias):
    return pl.pallas_call(
        _router_kernel,
        grid=(1,),
        in_specs=[
            pl.BlockSpec((T, H), lambda i: (0, 0)),
            pl.BlockSpec((E, H), lambda i: (0, 0)),
            pl.BlockSpec((1, E), lambda i: (0, 0)),
        ],
        out_specs=[
            pl.BlockSpec((T, 1), lambda i: (0, 0)),
            pl.BlockSpec((T, 1), lambda i: (0, 0)),
            pl.BlockSpec((T, 1), lambda i: (0, 0)),
            pl.BlockSpec((T, 1), lambda i: (0, 0)),
            pl.BlockSpec((NBLK, 1), lambda i: (0, 0)),
            pl.BlockSpec((NBLK, 1), lambda i: (0, 0)),
        ],
        out_shape=[
            jax.ShapeDtypeStruct((T, 1), jnp.int32),
            jax.ShapeDtypeStruct((T, 1), jnp.int32),
            jax.ShapeDtypeStruct((T, 1), jnp.float32),
            jax.ShapeDtypeStruct((T, 1), jnp.float32),
            jax.ShapeDtypeStruct((NBLK, 1), jnp.int32),
            jax.ShapeDtypeStruct((NBLK, 1), jnp.int32),
        ],
    )(x, gate_w, bias)


def _dispatch_body(x_hbm, pos1_hbm, pos2_hbm, w1_hbm, w2_hbm,
                   xs_hbm, ws_hbm,
                   rows_v, idx1_v, idx2_v, wv1, wv2, sem):
    wid = lax.axis_index("s") * NC + lax.axis_index("c")
    base = wid * TPT
    pltpu.sync_copy(pos1_hbm.at[pl.ds(base, TPT)], idx1_v)
    pltpu.sync_copy(pos2_hbm.at[pl.ds(base, TPT)], idx2_v)
    pltpu.sync_copy(w1_hbm.at[pl.ds(base, TPT)], wv1)
    pltpu.sync_copy(w2_hbm.at[pl.ds(base, TPT)], wv2)
    c3 = pltpu.async_copy(wv1, ws_hbm.at[idx1_v], sem)
    c4 = pltpu.async_copy(wv2, ws_hbm.at[idx2_v], sem)
    pltpu.sync_copy(x_hbm.at[pl.ds(base, TPT), :], rows_v)
    c1 = pltpu.async_copy(rows_v, xs_hbm.at[idx1_v], sem)
    c2 = pltpu.async_copy(rows_v, xs_hbm.at[idx2_v], sem)
    c3.wait()
    c4.wait()
    c1.wait()
    c2.wait()


def _dispatch(x, pos1, pos2, w1, w2):
    mesh = plsc.VectorSubcoreMesh(core_axis_name="c", subcore_axis_name="s",
                                  num_cores=NC, num_subcores=NS)
    fn = pl.kernel(
        _dispatch_body,
        out_type=[
            jax.ShapeDtypeStruct((PAD_LEN, H), jnp.float32),
            jax.ShapeDtypeStruct((PAD_LEN,), jnp.float32),
        ],
        mesh=mesh,
        scratch_types=[
            pltpu.VMEM((TPT, H), jnp.float32),
            pltpu.VMEM((TPT,), jnp.int32),
            pltpu.VMEM((TPT,), jnp.int32),
            pltpu.VMEM((TPT,), jnp.float32),
            pltpu.VMEM((TPT,), jnp.float32),
            pltpu.SemaphoreType.DMA,
        ],
    )
    return fn(x, pos1, pos2, w1, w2)


def _shared_kernel(x_ref, sgw_ref, suw_ref, sdw_ref, so_ref):
    xc = x_ref[...]
    sg = lax.dot_general(xc, sgw_ref[...], (((1,), (1,)), ((), ())),
                         preferred_element_type=jnp.float32)
    su = lax.dot_general(xc, suw_ref[...], (((1,), (1,)), ((), ())),
                         preferred_element_type=jnp.float32)
    hsh = jax.nn.silu(sg) * su
    so_ref[...] = lax.dot_general(hsh, sdw_ref[...], (((1,), (1,)), ((), ())),
                                  preferred_element_type=jnp.float32)


SHB = 256


def _shared(x, sgw, suw, sdw):
    return pl.pallas_call(
        _shared_kernel,
        grid=(T // SHB,),
        in_specs=[
            pl.BlockSpec((SHB, H), lambda b: (b, 0)),
            pl.BlockSpec((SI, H), lambda b: (0, 0)),
            pl.BlockSpec((SI, H), lambda b: (0, 0)),
            pl.BlockSpec((H, SI), lambda b: (0, 0)),
        ],
        out_specs=pl.BlockSpec((SHB, H), lambda b: (b, 0)),
        out_shape=jax.ShapeDtypeStruct((T, H), jnp.float32),
        compiler_params=pltpu.CompilerParams(
            dimension_semantics=("arbitrary",),
        ),
    )(x, sgw, suw, sdw)


def _grouped_kernel(be_ref, valid_ref, xs_ref, gup_ref, down_ref, ws_ref,
                    yw_ref):
    b = pl.program_id(0)

    @pl.when(valid_ref[b] > 0)
    def _():
        xb = xs_ref[...]
        gu = lax.dot_general(xb, gup_ref[0], (((1,), (1,)), ((), ())),
                             preferred_element_type=jnp.float32)
        h = jax.nn.silu(gu[:, :I]) * gu[:, I:]
        y = lax.dot_general(h, down_ref[0], (((1,), (1,)), ((), ())),
                            preferred_element_type=jnp.float32)
        w = ws_ref[0, 0, :].reshape(B, 1)
        yw_ref[...] = w * y


def _grouped(be, valid, xs, gup, down, ws3):
    grid_spec = pltpu.PrefetchScalarGridSpec(
        num_scalar_prefetch=2,
        grid=(NBLK,),
        in_specs=[
            pl.BlockSpec((B, H), lambda b, be_r, v_r: (b, 0)),
            pl.BlockSpec((1, 2 * I, H), lambda b, be_r, v_r: (be_r[b], 0, 0)),
            pl.BlockSpec((1, H, I), lambda b, be_r, v_r: (be_r[b], 0, 0)),
            pl.BlockSpec((1, 1, B), lambda b, be_r, v_r: (b, 0, 0)),
        ],
        out_specs=[
            pl.BlockSpec((B, H), lambda b, be_r, v_r: (b, 0)),
        ],
    )
    return pl.pallas_call(
        _grouped_kernel,
        grid_spec=grid_spec,
        out_shape=[
            jax.ShapeDtypeStruct((PAD_LEN, H), jnp.float32),
        ],
        compiler_params=pltpu.CompilerParams(
            dimension_semantics=("arbitrary",),
        ),
    )(be, valid, xs, gup, down, ws3)[0]


def _combine_body(yw_hbm, so_hbm, pos1_hbm, pos2_hbm, out_hbm,
                  y1_v, y2_v, sh_v, i1_v, i2_v, sem):
    wid = lax.axis_index("s") * NC + lax.axis_index("c")
    for ch in range(TPT // CT):
        tbase = wid * TPT + ch * CT
        pltpu.sync_copy(pos1_hbm.at[pl.ds(tbase, CT)], i1_v)
        pltpu.sync_copy(pos2_hbm.at[pl.ds(tbase, CT)], i2_v)
        g1 = pltpu.async_copy(yw_hbm.at[i1_v], y1_v, sem)
        g2 = pltpu.async_copy(yw_hbm.at[i2_v], y2_v, sem)
        pltpu.sync_copy(so_hbm.at[pl.ds(tbase, CT), :], sh_v)
        g1.wait()
        g2.wait()

        def body(r, _):
            for c in range(H // 16):
                sl = pl.ds(c * 16, 16)
                y1_v[r, sl] = y1_v[r, sl] + y2_v[r, sl] + sh_v[r, sl]
            return 0

        lax.fori_loop(0, CT, body, 0)
        pltpu.sync_copy(y1_v, out_hbm.at[pl.ds(tbase, CT), :])


def _combine(yw, so, pos1, pos2):
    mesh = plsc.VectorSubcoreMesh(core_axis_name="c", subcore_axis_name="s",
                                  num_cores=NC, num_subcores=NS)
    fn = pl.kernel(
        _combine_body,
        out_type=jax.ShapeDtypeStruct((T, H), jnp.float32),
        mesh=mesh,
        scratch_types=[
            pltpu.VMEM((CT, H), jnp.float32),
            pltpu.VMEM((CT, H), jnp.float32),
            pltpu.VMEM((CT, H), jnp.float32),
            pltpu.VMEM((CT,), jnp.int32),
            pltpu.VMEM((CT,), jnp.int32),
            pltpu.SemaphoreType.DMA,
        ],
    )
    return fn(yw, so, pos1, pos2)


def kernel(hidden_states, gate_w, e_score_correction_bias, gate_up_proj,
           down_proj, shared_gate_w, shared_up_w, shared_down_w):
    x = hidden_states.reshape(T, H)
    bias = e_score_correction_bias.reshape(1, E)

    pos1, pos2, w1, w2, be, valid = _router(x, gate_w, bias)
    pos1f = pos1.reshape(T)
    pos2f = pos2.reshape(T)

    xs, ws = _dispatch(x, pos1f, pos2f, w1.reshape(T), w2.reshape(T))
    so = _shared(x, shared_gate_w, shared_up_w, shared_down_w)

    yw = _grouped(be.reshape(NBLK), valid.reshape(NBLK), xs,
                  gate_up_proj, down_proj, ws.reshape(NBLK, 1, B))

    out = _combine(yw, so, pos1f, pos2f)
    return out.reshape(hidden_states.shape)


# speedup vs baseline: 1.4751x; 1.0162x over previous
import functools

import jax
import jax.numpy as jnp
from jax import lax
from jax.experimental import pallas as pl
from jax.experimental.pallas import tpu as pltpu
from jax.experimental.pallas import tpu_sc as plsc

T = 2048
H = 1024
I = 512
E = 16
SI = 1024
SCALE = 2.5
NEG_INF = -1e30

B = 256
NBLK = 32
PAD_LEN = NBLK * B
RCH = 256
SHC = T // NBLK

NC, NS = 2, 16
NW = NC * NS
TPT = T // NW
CT = 16


def _router_kernel(x_ref, gate_w_ref, bias_ref,
                   pos1_ref, pos2_ref, w1_ref, w2_ref, be_ref, valid_ref):
    x = x_ref[...]
    logits = lax.dot_general(x, gate_w_ref[...], (((1,), (1,)), ((), ())),
                             preferred_element_type=jnp.float32)
    scores = jax.nn.sigmoid(logits)
    sc = scores + bias_ref[...]
    iota = lax.broadcasted_iota(jnp.int32, (T, E), 1)
    m1 = jnp.max(sc, axis=-1, keepdims=True)
    idx1 = jnp.min(jnp.where(sc == m1, iota, E), axis=-1, keepdims=True)
    sc2 = jnp.where(iota == idx1, NEG_INF, sc)
    m2 = jnp.max(sc2, axis=-1, keepdims=True)
    idx2 = jnp.min(jnp.where(sc2 == m2, iota, E), axis=-1, keepdims=True)
    w1 = jnp.sum(jnp.where(iota == idx1, scores, 0.0), axis=-1, keepdims=True)
    w2 = jnp.sum(jnp.where(iota == idx2, scores, 0.0), axis=-1, keepdims=True)
    denom = w1 + w2 + 1e-20
    w1_ref[...] = w1 / denom * SCALE
    w2_ref[...] = w2 / denom * SCALE

    oh1 = (iota == idx1).astype(jnp.float32)
    oh2 = (iota == idx2).astype(jnp.float32)

    tri = (lax.broadcasted_iota(jnp.int32, (RCH, RCH), 0) >
           lax.broadcasted_iota(jnp.int32, (RCH, RCH), 1)).astype(jnp.float32)
    off = jnp.zeros((1, E), jnp.float32)
    ranks = []
    for oh in (oh1, oh2):
        rs = []
        for c in range(T // RCH):
            ohc = oh[c * RCH:(c + 1) * RCH]
            exc = lax.dot_general(tri, ohc, (((1,), (0,)), ((), ())),
                                  preferred_element_type=jnp.float32) + off
            rs.append(jnp.sum(exc * ohc, axis=-1, keepdims=True))
            off = off + jnp.sum(ohc, axis=0, keepdims=True)
        ranks.append(jnp.concatenate(rs, axis=0))
    counts = off

    pc_pad = jnp.ceil(counts / B) * B
    mstrict = (lax.broadcasted_iota(jnp.int32, (E, E), 0) <
               lax.broadcasted_iota(jnp.int32, (E, E), 1)).astype(jnp.float32)
    pad_off = lax.dot_general(pc_pad, mstrict, (((1,), (0,)), ((), ())),
                              preferred_element_type=jnp.float32)
    total_pad = jnp.sum(pc_pad, axis=-1, keepdims=True)

    sel1 = jnp.sum(oh1 * pad_off, axis=-1, keepdims=True)
    sel2 = jnp.sum(oh2 * pad_off, axis=-1, keepdims=True)
    pos1_ref[...] = (sel1 + ranks[0]).astype(jnp.int32)
    pos2_ref[...] = (sel2 + ranks[1]).astype(jnp.int32)

    bb = lax.broadcasted_iota(jnp.int32, (NBLK, E), 0).astype(jnp.float32) * B
    le = (jnp.broadcast_to(pad_off, (NBLK, E)) <= bb).astype(jnp.float32)
    be_ref[...] = (jnp.sum(le, axis=-1, keepdims=True) - 1.0).astype(jnp.int32)
    bb0 = lax.broadcasted_iota(jnp.int32, (NBLK, 1), 0).astype(jnp.float32) * B
    valid_ref[...] = (bb0 < total_pad).astype(jnp.int32)


def _router(x, gate_w, bias):
    return pl.pallas_call(
        _router_kernel,
        grid=(1,),
        in_specs=[
            pl.BlockSpec((T, H), lambda i: (0, 0)),
            pl.BlockSpec((E, H), lambda i: (0, 0)),
            pl.BlockSpec((1, E), lambda i: (0, 0)),
        ],
        out_specs=[
            pl.BlockSpec((T, 1), lambda i: (0, 0)),
            pl.BlockSpec((T, 1), lambda i: (0, 0)),
            pl.BlockSpec((T, 1), lambda i: (0, 0)),
            pl.BlockSpec((T, 1), lambda i: (0, 0)),
            pl.BlockSpec((NBLK, 1), lambda i: (0, 0)),
            pl.BlockSpec((NBLK, 1), lambda i: (0, 0)),
        ],
        out_shape=[
            jax.ShapeDtypeStruct((T, 1), jnp.int32),
            jax.ShapeDtypeStruct((T, 1), jnp.int32),
            jax.ShapeDtypeStruct((T, 1), jnp.float32),
            jax.ShapeDtypeStruct((T, 1), jnp.float32),
            jax.ShapeDtypeStruct((NBLK, 1), jnp.int32),
            jax.ShapeDtypeStruct((NBLK, 1), jnp.int32),
        ],
    )(x, gate_w, bias)


DCH = 4
DR = TPT // DCH


def _dispatch_body(x_hbm, pos1_hbm, pos2_hbm, w1_hbm, w2_hbm,
                   xs_hbm, ws_hbm,
                   rows_v, idx1_v, idx2_v, wv1, wv2, lsem, ssem):
    wid = lax.axis_index("s") * NC + lax.axis_index("c")
    base = wid * TPT
    for c in range(DCH):
        pltpu.sync_copy(pos1_hbm.at[pl.ds(base + c * DR, DR)], idx1_v.at[c])
        pltpu.sync_copy(pos2_hbm.at[pl.ds(base + c * DR, DR)], idx2_v.at[c])
    pltpu.sync_copy(w1_hbm.at[pl.ds(base, TPT)], wv1)
    pltpu.sync_copy(w2_hbm.at[pl.ds(base, TPT)], wv2)
    loads = [pltpu.async_copy(x_hbm.at[pl.ds(base + c * DR, DR), :],
                              rows_v.at[pl.ds(c * DR, DR), :], lsem)
             for c in range(DCH)]
    scat = []
    for c in range(DCH):
        loads[c].wait()
        rv = rows_v.at[pl.ds(c * DR, DR), :]
        scat.append(pltpu.async_copy(rv, xs_hbm.at[idx1_v.at[c]], ssem))
        scat.append(pltpu.async_copy(rv, xs_hbm.at[idx2_v.at[c]], ssem))
        wv1c = wv1.at[pl.ds(c * DR, DR)]
        wv2c = wv2.at[pl.ds(c * DR, DR)]
        scat.append(pltpu.async_copy(wv1c, ws_hbm.at[idx1_v.at[c]], ssem))
        scat.append(pltpu.async_copy(wv2c, ws_hbm.at[idx2_v.at[c]], ssem))
    for s in scat:
        s.wait()


def _dispatch(x, pos1, pos2, w1, w2):
    mesh = plsc.VectorSubcoreMesh(core_axis_name="c", subcore_axis_name="s",
                                  num_cores=NC, num_subcores=NS)
    fn = pl.kernel(
        _dispatch_body,
        out_type=[
            jax.ShapeDtypeStruct((PAD_LEN, H), jnp.float32),
            jax.ShapeDtypeStruct((PAD_LEN,), jnp.float32),
        ],
        mesh=mesh,
        scratch_types=[
            pltpu.VMEM((TPT, H), jnp.float32),
            pltpu.VMEM((DCH, DR), jnp.int32),
            pltpu.VMEM((DCH, DR), jnp.int32),
            pltpu.VMEM((TPT,), jnp.float32),
            pltpu.VMEM((TPT,), jnp.float32),
            pltpu.SemaphoreType.DMA,
            pltpu.SemaphoreType.DMA,
        ],
    )
    return fn(x, pos1, pos2, w1, w2)


def _shared_kernel(x_ref, sgw_ref, suw_ref, sdw_ref, so_ref):
    xc = x_ref[...]
    sg = lax.dot_general(xc, sgw_ref[...], (((1,), (1,)), ((), ())),
                         preferred_element_type=jnp.float32)
    su = lax.dot_general(xc, suw_ref[...], (((1,), (1,)), ((), ())),
                         preferred_element_type=jnp.float32)
    hsh = jax.nn.silu(sg) * su
    so_ref[...] = lax.dot_general(hsh, sdw_ref[...], (((1,), (1,)), ((), ())),
                                  preferred_element_type=jnp.float32)


SHB = 256


def _shared(x, sgw, suw, sdw):
    return pl.pallas_call(
        _shared_kernel,
        grid=(T // SHB,),
        in_specs=[
            pl.BlockSpec((SHB, H), lambda b: (b, 0)),
            pl.BlockSpec((SI, H), lambda b: (0, 0)),
            pl.BlockSpec((SI, H), lambda b: (0, 0)),
            pl.BlockSpec((H, SI), lambda b: (0, 0)),
        ],
        out_specs=pl.BlockSpec((SHB, H), lambda b: (b, 0)),
        out_shape=jax.ShapeDtypeStruct((T, H), jnp.float32),
        compiler_params=pltpu.CompilerParams(
            dimension_semantics=("arbitrary",),
        ),
    )(x, sgw, suw, sdw)


def _grouped_kernel(be_ref, valid_ref, xs_ref, gup_ref, down_ref, ws_ref,
                    yw_ref):
    b = pl.program_id(0)

    @pl.when(valid_ref[b] > 0)
    def _():
        xb = xs_ref[...]
        gu = lax.dot_general(xb, gup_ref[0], (((1,), (1,)), ((), ())),
                             preferred_element_type=jnp.float32)
        h = jax.nn.silu(gu[:, :I]) * gu[:, I:]
        y = lax.dot_general(h, down_ref[0], (((1,), (1,)), ((), ())),
                            preferred_element_type=jnp.float32)
        w = ws_ref[0, 0, :].reshape(B, 1)
        yw_ref[...] = w * y


def _grouped(be, valid, xs, gup, down, ws3):
    grid_spec = pltpu.PrefetchScalarGridSpec(
        num_scalar_prefetch=2,
        grid=(NBLK,),
        in_specs=[
            pl.BlockSpec((B, H), lambda b, be_r, v_r: (b, 0)),
            pl.BlockSpec((1, 2 * I, H), lambda b, be_r, v_r: (be_r[b], 0, 0)),
            pl.BlockSpec((1, H, I), lambda b, be_r, v_r: (be_r[b], 0, 0)),
            pl.BlockSpec((1, 1, B), lambda b, be_r, v_r: (b, 0, 0)),
        ],
        out_specs=[
            pl.BlockSpec((B, H), lambda b, be_r, v_r: (b, 0)),
        ],
    )
    return pl.pallas_call(
        _grouped_kernel,
        grid_spec=grid_spec,
        out_shape=[
            jax.ShapeDtypeStruct((PAD_LEN, H), jnp.float32),
        ],
        compiler_params=pltpu.CompilerParams(
            dimension_semantics=("arbitrary",),
        ),
    )(be, valid, xs, gup, down, ws3)[0]


CCH = TPT // CT


def _combine_body(yw_hbm, so_hbm, pos1_hbm, pos2_hbm, out_hbm,
                  y1_v, y2_v, sh_v, i1_v, i2_v, gsem, osem):
    wid = lax.axis_index("s") * NC + lax.axis_index("c")
    tb = wid * TPT
    pltpu.sync_copy(pos1_hbm.at[pl.ds(tb, TPT)], i1_v)
    pltpu.sync_copy(pos2_hbm.at[pl.ds(tb, TPT)], i2_v)

    descs = {}
    stores = [None, None]

    def issue(ch):
        slot = ch & 1
        if stores[slot] is not None:
            stores[slot].wait()
            stores[slot] = None
        sl = pl.ds(ch * CT, CT)
        descs[ch] = (
            pltpu.async_copy(yw_hbm.at[i1_v.at[sl]], y1_v.at[slot], gsem),
            pltpu.async_copy(yw_hbm.at[i2_v.at[sl]], y2_v.at[slot], gsem),
            pltpu.async_copy(so_hbm.at[pl.ds(tb + ch * CT, CT), :],
                             sh_v.at[slot], gsem),
        )

    issue(0)
    for ch in range(CCH):
        slot = ch & 1
        if ch + 1 < CCH:
            issue(ch + 1)
        for d in descs[ch]:
            d.wait()

        def body(r, _):
            for c in range(H // 16):
                sl = pl.ds(c * 16, 16)
                y1_v[slot, r, sl] = (y1_v[slot, r, sl] + y2_v[slot, r, sl]
                                     + sh_v[slot, r, sl])
            return 0

        lax.fori_loop(0, CT, body, 0)
        stores[slot] = pltpu.async_copy(
            y1_v.at[slot], out_hbm.at[pl.ds(tb + ch * CT, CT), :], osem)
    for s in stores:
        if s is not None:
            s.wait()


def _combine(yw, so, pos1, pos2):
    mesh = plsc.VectorSubcoreMesh(core_axis_name="c", subcore_axis_name="s",
                                  num_cores=NC, num_subcores=NS)
    fn = pl.kernel(
        _combine_body,
        out_type=jax.ShapeDtypeStruct((T, H), jnp.float32),
        mesh=mesh,
        scratch_types=[
            pltpu.VMEM((2, CT, H), jnp.float32),
            pltpu.VMEM((2, CT, H), jnp.float32),
            pltpu.VMEM((2, CT, H), jnp.float32),
            pltpu.VMEM((TPT,), jnp.int32),
            pltpu.VMEM((TPT,), jnp.int32),
            pltpu.SemaphoreType.DMA,
            pltpu.SemaphoreType.DMA,
        ],
    )
    return fn(yw, so, pos1, pos2)


def kernel(hidden_states, gate_w, e_score_correction_bias, gate_up_proj,
           down_proj, shared_gate_w, shared_up_w, shared_down_w):
    x = hidden_states.reshape(T, H)
    bias = e_score_correction_bias.reshape(1, E)

    pos1, pos2, w1, w2, be, valid = _router(x, gate_w, bias)
    pos1f = pos1.reshape(T)
    pos2f = pos2.reshape(T)

    xs, ws = _dispatch(x, pos1f, pos2f, w1.reshape(T), w2.reshape(T))
    so = _shared(x, shared_gate_w, shared_up_w, shared_down_w)

    yw = _grouped(be.reshape(NBLK), valid.reshape(NBLK), xs,
                  gate_up_proj, down_proj, ws.reshape(NBLK, 1, B))

    out = _combine(yw, so, pos1f, pos2f)
    return out.reshape(hidden_states.shape)
